# baseline (speedup 1.0000x reference)
import functools

import jax
import jax.numpy as jnp
from jax import lax
from jax.experimental import pallas as pl
from jax.experimental.pallas import tpu as pltpu
from jax.experimental.pallas import tpu_sc as plsc

_N = 10000
_D = 128
_E = 320000

_NW = 32
_C = 128
_K = 80
_EPAD = _NW * _K * _C
_NPAD = 10112
_RPW = _NPAD // 16

_f32 = jnp.float32


def _sc_spmm():
    scratch = [
        pltpu.VMEM((_K, _C), jnp.int32),
        pltpu.VMEM((2, _C), jnp.int32),
        pltpu.VMEM((2, _C), jnp.int32),
        pltpu.VMEM((2, _C, _D), _f32),
        pltpu.VMEM_SHARED((_NPAD, _D), _f32),
        pltpu.SemaphoreType.DMA,
        pltpu.SemaphoreType.DMA,
    ]

    def body(h_hbm, pk_hbm, agg_out, pk_v, sidx, didx, rb, agg_sh,
             gsem0, gsem1):
        c = lax.axis_index("c")
        s = lax.axis_index("s")
        wid = s * 2 + c
        base = s * _RPW
        gsems = (gsem0, gsem1)

        pltpu.sync_copy(pk_hbm.at[wid], pk_v)

        def zrow(i, carry):
            for j in range(_D // 16):
                rb[0, i, pl.ds(j * 16, 16)] = jnp.zeros((16,), _f32)
            return carry
        lax.fori_loop(0, _C, zrow, 0)

        def zagg(k, carry):
            pltpu.sync_copy(rb.at[0], agg_sh.at[pl.ds(base + k * _C, _C)])
            return carry
        lax.fori_loop(0, _RPW // _C, zagg, 0)
        rem = _RPW % _C
        if rem:
            pltpu.sync_copy(rb.at[0, pl.ds(0, rem)],
                            agg_sh.at[pl.ds(base + _RPW - rem, rem)])

        plsc.subcore_barrier()

        def unpack(g, b):
            for j in range(_C // 16):
                v = pk_v[g, pl.ds(j * 16, 16)]
                sidx[b, pl.ds(j * 16, 16)] = v & 0xFFFF
                didx[b, pl.ds(j * 16, 16)] = v >> 16

        def start_gather(b):
            pltpu.async_copy(h_hbm.at[sidx.at[b]], rb.at[b], gsems[b])

        def wait_gather(b):
            pltpu.make_async_copy(h_hbm.at[sidx.at[b]], rb.at[b],
                                  gsems[b]).wait()

        def scatter(b):
            pltpu.sync_copy(rb.at[b], agg_sh.at[didx.at[b]], add=True)

        for b in (0, 1):
            unpack(b, b)
            start_gather(b)

        def pair(p, carry):
            g = p * 2
            for b in (0, 1):
                wait_gather(b)
                scatter(b)
                unpack(g + b + 2, b)
                start_gather(b)
            return carry
        lax.fori_loop(0, (_K - 2) // 2, pair, 0)

        for b in (0, 1):
            wait_gather(b)
            scatter(b)

        plsc.subcore_barrier()

        pltpu.sync_copy(agg_sh.at[pl.ds(base, _RPW)],
                        agg_out.at[c, pl.ds(base, _RPW)])

    return pl.kernel(
        body,
        out_type=jax.ShapeDtypeStruct((2, _NPAD, _D), _f32),
        mesh=plsc.VectorSubcoreMesh(core_axis_name="c", subcore_axis_name="s"),
        scratch_types=scratch,
    )


def _sc_spmm_deg():
    scratch = [
        pltpu.VMEM((_K, _C), jnp.int32),
        pltpu.VMEM((2, _C), jnp.int32),
        pltpu.VMEM((2, _C), jnp.int32),
        pltpu.VMEM((2, _C, _D), _f32),
        pltpu.VMEM_SHARED((_NPAD, _D), _f32),
        pltpu.SemaphoreType.DMA,
        pltpu.SemaphoreType.DMA,
    ]

    def body(h_hbm, pk_hbm, agg_out, deg_out, pk_v, sidx, didx, rb, agg_sh,
             gsem0, gsem1):
        c = lax.axis_index("c")
        s = lax.axis_index("s")
        wid = s * 2 + c
        base = s * _RPW
        gsems = (gsem0, gsem1)

        pltpu.sync_copy(pk_hbm.at[wid], pk_v)

        def z01(i, carry):
            for j in range(_D // 16):
                rb[0, i, pl.ds(j * 16, 16)] = jnp.zeros((16,), _f32)
                rb[1, i, pl.ds(j * 16, 16)] = jnp.ones((16,), _f32)
            return carry
        lax.fori_loop(0, _C, z01, 0)

        def zagg(k, carry):
            pltpu.sync_copy(rb.at[0], agg_sh.at[pl.ds(base + k * _C, _C)])
            return carry

        def zero_acc():
            lax.fori_loop(0, _RPW // _C, zagg, 0)
            rem = _RPW % _C
            if rem:
                pltpu.sync_copy(rb.at[0, pl.ds(0, rem)],
                                agg_sh.at[pl.ds(base + _RPW - rem, rem)])

        zero_acc()
        plsc.subcore_barrier()

        def unpack_dst(g, carry):
            for j in range(_C // 16):
                didx[0, pl.ds(j * 16, 16)] = pk_v[g, pl.ds(j * 16, 16)] >> 16
            return carry

        def deg_chunk(g, carry):
            unpack_dst(g, carry)
            pltpu.sync_copy(rb.at[1], agg_sh.at[didx.at[0]], add=True)
            return carry
        lax.fori_loop(0, _K, deg_chunk, 0)

        plsc.subcore_barrier()
        pltpu.sync_copy(agg_sh.at[pl.ds(base, _RPW)],
                        deg_out.at[c, pl.ds(base, _RPW)])
        zero_acc()
        plsc.subcore_barrier()

        def unpack(g, b):
            for j in range(_C // 16):
                v = pk_v[g, pl.ds(j * 16, 16)]
                sidx[b, pl.ds(j * 16, 16)] = v & 0xFFFF
                didx[b, pl.ds(j * 16, 16)] = v >> 16

        def start_gather(b):
            pltpu.async_copy(h_hbm.at[sidx.at[b]], rb.at[b], gsems[b])

        def wait_gather(b):
            pltpu.make_async_copy(h_hbm.at[sidx.at[b]], rb.at[b],
                                  gsems[b]).wait()

        def scatter(b):
            pltpu.sync_copy(rb.at[b], agg_sh.at[didx.at[b]], add=True)

        for b in (0, 1):
            unpack(b, b)
            start_gather(b)

        def pair(p, carry):
            g = p * 2
            for b in (0, 1):
                wait_gather(b)
                scatter(b)
                unpack(g + b + 2, b)
                start_gather(b)
            return carry
        lax.fori_loop(0, (_K - 2) // 2, pair, 0)

        for b in (0, 1):
            wait_gather(b)
            scatter(b)

        plsc.subcore_barrier()
        pltpu.sync_copy(agg_sh.at[pl.ds(base, _RPW)],
                        agg_out.at[c, pl.ds(base, _RPW)])

    return pl.kernel(
        body,
        out_type=(jax.ShapeDtypeStruct((2, _NPAD, _D), _f32),
                  jax.ShapeDtypeStruct((2, _NPAD, _D), _f32)),
        mesh=plsc.VectorSubcoreMesh(core_axis_name="c", subcore_axis_name="s"),
        scratch_types=scratch,
    )


_GRID = 25
_BR = _N // _GRID


def _row_spec():
    return pl.BlockSpec((_BR, _D), lambda i: (i, 0))


def _w_spec():
    return pl.BlockSpec((_D, _D), lambda i: (0, 0))


def _b_spec():
    return pl.BlockSpec((1, _D), lambda i: (0, 0))


def _agg_spec():
    return pl.BlockSpec((2, _BR, _D), lambda i: (0, i, 0))


def _deg_spec():
    return pl.BlockSpec((2, _BR, _D), lambda i: (0, i, 0))


def _tc_h(x, W1, b1):
    def body(xr, w1r, b1r, hr):
        hr[...] = jnp.dot(xr[...], w1r[...],
                          preferred_element_type=_f32) + b1r[...]

    return pl.pallas_call(
        body,
        grid=(_GRID,),
        in_specs=[_row_spec(), _w_spec(), _b_spec()],
        out_specs=_row_spec(),
        out_shape=jax.ShapeDtypeStruct((_N, _D), _f32),
    )(x, W1, b1.reshape(1, _D))


def _tc_y0(x, W0, b0):
    def body(xr, w0r, b0r, y0r):
        y0r[...] = jnp.dot(xr[...], w0r[...],
                           preferred_element_type=_f32) + b0r[...]

    return pl.pallas_call(
        body,
        grid=(_GRID,),
        in_specs=[_row_spec(), _w_spec(), _b_spec()],
        out_specs=_row_spec(),
        out_shape=jax.ShapeDtypeStruct((_N, _D), _f32),
    )(x, W0, b0.reshape(1, _D))


def _tc_comb(relu, y0, aggp, degp, W1, b1):
    def body(y0r, aggr, degr, w1r, b1r, xo, hn):
        av = aggr[...]
        agg = av[0] + av[1]
        dv = degr[...]
        deg = dv[0, :, 0:1] + dv[1, :, 0:1]
        x = y0r[...] + agg / jnp.maximum(deg, 1.0)
        if relu:
            x = jnp.maximum(x, 0.0)
        xo[...] = x
        hn[...] = jnp.dot(x, w1r[...], preferred_element_type=_f32) + b1r[...]

    return pl.pallas_call(
        body,
        grid=(_GRID,),
        in_specs=[_row_spec(), _agg_spec(), _deg_spec(),
                  _w_spec(), _b_spec()],
        out_specs=(_row_spec(), _row_spec()),
        out_shape=(jax.ShapeDtypeStruct((_N, _D), _f32),
                   jax.ShapeDtypeStruct((_N, _D), _f32)),
    )(y0, aggp, degp, W1, b1.reshape(1, _D))


def _tc_final(y0, aggp, degp, res):
    def body(y0r, aggr, degr, resr, outr):
        av = aggr[...]
        agg = av[0] + av[1]
        dv = degr[...]
        deg = dv[0, :, 0:1] + dv[1, :, 0:1]
        x = y0r[...] + agg / jnp.maximum(deg, 1.0) + resr[...]
        outr[...] = jnp.maximum(x, 0.0)

    return pl.pallas_call(
        body,
        grid=(_GRID,),
        in_specs=[_row_spec(), _agg_spec(), _deg_spec(), _row_spec()],
        out_specs=_row_spec(),
        out_shape=jax.ShapeDtypeStruct((_N, _D), _f32),
    )(y0, aggp, degp, res)


@jax.jit
def kernel(features, edges, dis, W0_0, b0_0, W1_0, b1_0,
           W0_1, b0_1, W1_1, b1_1, W0_2, b0_2, W1_2, b1_2):
    del dis
    src = edges[0]
    dst = edges[1]
    padlen = _EPAD - _E
    ar = jnp.arange(padlen, dtype=jnp.int32)
    src_p = jnp.concatenate([src, ar % _N])
    dst_p = jnp.concatenate([dst, _N + (ar % (_NPAD - _N))])
    pk_r = (src_p + dst_p * 65536).reshape(_NW, _K, _C)

    h0 = _tc_h(features, W1_0, b1_0)
    aggp0, degp = _sc_spmm_deg()(h0, pk_r)
    y00 = _tc_y0(features, W0_0, b0_0)
    x1, h1 = _tc_comb(False, y00, aggp0, degp, W1_1, b1_1)
    aggp1 = _sc_spmm()(h1, pk_r)
    y01 = _tc_y0(x1, W0_1, b0_1)
    x2, h2 = _tc_comb(True, y01, aggp1, degp, W1_2, b1_2)
    aggp2 = _sc_spmm()(h2, pk_r)
    y02 = _tc_y0(x2, W0_2, b0_2)
    return _tc_final(y02, aggp2, degp, features)


# speedup vs baseline: 1.0393x; 1.0393x over previous
import functools

import jax
import jax.numpy as jnp
from jax import lax
from jax.experimental import pallas as pl
from jax.experimental.pallas import tpu as pltpu
from jax.experimental.pallas import tpu_sc as plsc

_N = 10000
_D = 128
_E = 320000

_NW = 32
_C = 128
_K = 80
_EPAD = _NW * _K * _C
_NPAD = 10112
_RPW = _NPAD // 16

_f32 = jnp.float32


def _sc_spmm():
    scratch = [
        pltpu.VMEM((_K, _C), jnp.int32),
        pltpu.VMEM((2, _C), jnp.int32),
        pltpu.VMEM((2, _C), jnp.int32),
        pltpu.VMEM((2, _C, _D), _f32),
        pltpu.VMEM_SHARED((_NPAD, _D), _f32),
        pltpu.SemaphoreType.DMA,
        pltpu.SemaphoreType.DMA,
    ]

    def body(h_hbm, pk_hbm, agg_out, pk_v, sidx, didx, rb, agg_sh,
             gsem0, gsem1):
        c = lax.axis_index("c")
        s = lax.axis_index("s")
        wid = s * 2 + c
        base = s * _RPW
        gsems = (gsem0, gsem1)

        pltpu.sync_copy(pk_hbm.at[wid], pk_v)

        def zrow(i, carry):
            for j in range(_D // 16):
                rb[0, i, pl.ds(j * 16, 16)] = jnp.zeros((16,), _f32)
            return carry
        lax.fori_loop(0, _C, zrow, 0)

        def zagg(k, carry):
            pltpu.sync_copy(rb.at[0], agg_sh.at[pl.ds(base + k * _C, _C)])
            return carry
        lax.fori_loop(0, _RPW // _C, zagg, 0)
        rem = _RPW % _C
        if rem:
            pltpu.sync_copy(rb.at[0, pl.ds(0, rem)],
                            agg_sh.at[pl.ds(base + _RPW - rem, rem)])

        plsc.subcore_barrier()

        def unpack(g, b):
            for j in range(_C // 16):
                v = pk_v[g, pl.ds(j * 16, 16)]
                sidx[b, pl.ds(j * 16, 16)] = v & 0xFFFF
                didx[b, pl.ds(j * 16, 16)] = v >> 16

        def start_gather(b):
            pltpu.async_copy(h_hbm.at[sidx.at[b]], rb.at[b], gsems[b])

        def wait_gather(b):
            pltpu.make_async_copy(h_hbm.at[sidx.at[b]], rb.at[b],
                                  gsems[b]).wait()

        def scatter(b):
            pltpu.sync_copy(rb.at[b], agg_sh.at[didx.at[b]], add=True)

        for b in (0, 1):
            unpack(b, b)
            start_gather(b)

        def pair(p, carry):
            g = p * 2
            for b in (0, 1):
                wait_gather(b)
                scatter(b)
                unpack(g + b + 2, b)
                start_gather(b)
            return carry
        lax.fori_loop(0, (_K - 2) // 2, pair, 0)

        for b in (0, 1):
            wait_gather(b)
            scatter(b)

        plsc.subcore_barrier()

        pltpu.sync_copy(agg_sh.at[pl.ds(base, _RPW)],
                        agg_out.at[c, pl.ds(base, _RPW)])

    return pl.kernel(
        body,
        out_type=jax.ShapeDtypeStruct((2, _NPAD, _D), _f32),
        mesh=plsc.VectorSubcoreMesh(core_axis_name="c", subcore_axis_name="s"),
        scratch_types=scratch,
    )


def _sc_spmm_deg():
    scratch = [
        pltpu.VMEM((_K, _C), jnp.int32),
        pltpu.VMEM((2, _C), jnp.int32),
        pltpu.VMEM((2, _C), jnp.int32),
        pltpu.VMEM((2, _C, _D), _f32),
        pltpu.VMEM_SHARED((_NPAD, _D), _f32),
        pltpu.SemaphoreType.DMA,
        pltpu.SemaphoreType.DMA,
    ]

    def body(h_hbm, pk_hbm, agg_out, deg_out, pk_v, sidx, didx, rb, agg_sh,
             gsem0, gsem1):
        c = lax.axis_index("c")
        s = lax.axis_index("s")
        wid = s * 2 + c
        base = s * _RPW
        gsems = (gsem0, gsem1)

        pltpu.sync_copy(pk_hbm.at[wid], pk_v)

        def z01(i, carry):
            for j in range(_D // 16):
                rb[0, i, pl.ds(j * 16, 16)] = jnp.zeros((16,), _f32)
                rb[1, i, pl.ds(j * 16, 16)] = jnp.ones((16,), _f32)
            return carry
        lax.fori_loop(0, _C, z01, 0)

        def zagg(k, carry):
            pltpu.sync_copy(rb.at[0], agg_sh.at[pl.ds(base + k * _C, _C)])
            return carry

        def zero_acc():
            lax.fori_loop(0, _RPW // _C, zagg, 0)
            rem = _RPW % _C
            if rem:
                pltpu.sync_copy(rb.at[0, pl.ds(0, rem)],
                                agg_sh.at[pl.ds(base + _RPW - rem, rem)])

        zero_acc()
        plsc.subcore_barrier()

        def unpack_dst(g, carry):
            for j in range(_C // 16):
                didx[0, pl.ds(j * 16, 16)] = pk_v[g, pl.ds(j * 16, 16)] >> 16
            return carry

        def deg_chunk(g, carry):
            unpack_dst(g, carry)
            pltpu.sync_copy(rb.at[1], agg_sh.at[didx.at[0]], add=True)
            return carry
        lax.fori_loop(0, _K, deg_chunk, 0)

        plsc.subcore_barrier()
        pltpu.sync_copy(agg_sh.at[pl.ds(base, _RPW)],
                        deg_out.at[c, pl.ds(base, _RPW)])
        zero_acc()
        plsc.subcore_barrier()

        def unpack(g, b):
            for j in range(_C // 16):
                v = pk_v[g, pl.ds(j * 16, 16)]
                sidx[b, pl.ds(j * 16, 16)] = v & 0xFFFF
                didx[b, pl.ds(j * 16, 16)] = v >> 16

        def start_gather(b):
            pltpu.async_copy(h_hbm.at[sidx.at[b]], rb.at[b], gsems[b])

        def wait_gather(b):
            pltpu.make_async_copy(h_hbm.at[sidx.at[b]], rb.at[b],
                                  gsems[b]).wait()

        def scatter(b):
            pltpu.sync_copy(rb.at[b], agg_sh.at[didx.at[b]], add=True)

        for b in (0, 1):
            unpack(b, b)
            start_gather(b)

        def pair(p, carry):
            g = p * 2
            for b in (0, 1):
                wait_gather(b)
                scatter(b)
                unpack(g + b + 2, b)
                start_gather(b)
            return carry
        lax.fori_loop(0, (_K - 2) // 2, pair, 0)

        for b in (0, 1):
            wait_gather(b)
            scatter(b)

        plsc.subcore_barrier()
        pltpu.sync_copy(agg_sh.at[pl.ds(base, _RPW)],
                        agg_out.at[c, pl.ds(base, _RPW)])

    return pl.kernel(
        body,
        out_type=(jax.ShapeDtypeStruct((2, _NPAD, _D), _f32),
                  jax.ShapeDtypeStruct((2, _NPAD, _D), _f32)),
        mesh=plsc.VectorSubcoreMesh(core_axis_name="c", subcore_axis_name="s"),
        scratch_types=scratch,
    )


_GRID = 25
_BR = _N // _GRID


def _row_spec():
    return pl.BlockSpec((_BR, _D), lambda i: (i, 0))


def _w_spec():
    return pl.BlockSpec((_D, _D), lambda i: (0, 0))


def _b_spec():
    return pl.BlockSpec((1, _D), lambda i: (0, 0))


def _agg_spec():
    return pl.BlockSpec((2, _BR, _D), lambda i: (0, i, 0))


def _deg_spec():
    return pl.BlockSpec((2, _BR, _D), lambda i: (0, i, 0))


def _tc_y0(x, W0, b0):
    def body(xr, w0r, b0r, y0r):
        y0r[...] = jnp.dot(xr[...], w0r[...],
                           preferred_element_type=_f32) + b0r[...]

    return pl.pallas_call(
        body,
        grid=(_GRID,),
        in_specs=[_row_spec(), _w_spec(), _b_spec()],
        out_specs=_row_spec(),
        out_shape=jax.ShapeDtypeStruct((_N, _D), _f32),
    )(x, W0, b0.reshape(1, _D))


def _tc_comb(relu, y0, aggp, degp, W1, b1):
    def body(y0r, aggr, degr, w1r, b1r, xo):
        av = aggr[...]
        agg = av[0] + av[1]
        dv = degr[...]
        deg = dv[0, :, 0:1] + dv[1, :, 0:1]
        m = agg / jnp.maximum(deg, 1.0)
        x = (y0r[...] + jnp.dot(m, w1r[...], preferred_element_type=_f32)
             + b1r[...] * jnp.where(deg > 0.0, 1.0, 0.0))
        if relu:
            x = jnp.maximum(x, 0.0)
        xo[...] = x

    return pl.pallas_call(
        body,
        grid=(_GRID,),
        in_specs=[_row_spec(), _agg_spec(), _deg_spec(),
                  _w_spec(), _b_spec()],
        out_specs=_row_spec(),
        out_shape=jax.ShapeDtypeStruct((_N, _D), _f32),
    )(y0, aggp, degp, W1, b1.reshape(1, _D))


def _tc_final(y0, aggp, degp, W1, b1, res):
    def body(y0r, aggr, degr, w1r, b1r, resr, outr):
        av = aggr[...]
        agg = av[0] + av[1]
        dv = degr[...]
        deg = dv[0, :, 0:1] + dv[1, :, 0:1]
        m = agg / jnp.maximum(deg, 1.0)
        x = (y0r[...] + jnp.dot(m, w1r[...], preferred_element_type=_f32)
             + b1r[...] * jnp.where(deg > 0.0, 1.0, 0.0) + resr[...])
        outr[...] = jnp.maximum(x, 0.0)

    return pl.pallas_call(
        body,
        grid=(_GRID,),
        in_specs=[_row_spec(), _agg_spec(), _deg_spec(),
                  _w_spec(), _b_spec(), _row_spec()],
        out_specs=_row_spec(),
        out_shape=jax.ShapeDtypeStruct((_N, _D), _f32),
    )(y0, aggp, degp, W1, b1.reshape(1, _D), res)


@jax.jit
def kernel(features, edges, dis, W0_0, b0_0, W1_0, b1_0,
           W0_1, b0_1, W1_1, b1_1, W0_2, b0_2, W1_2, b1_2):
    del dis
    src = edges[0]
    dst = edges[1]
    padlen = _EPAD - _E
    ar = jnp.arange(padlen, dtype=jnp.int32)
    src_p = jnp.concatenate([src, ar % _N])
    dst_p = jnp.concatenate([dst, _N + (ar % (_NPAD - _N))])
    pk_r = (src_p + dst_p * 65536).reshape(_NW, _K, _C)

    aggp0, degp = _sc_spmm_deg()(features, pk_r)
    y00 = _tc_y0(features, W0_0, b0_0)
    x1 = _tc_comb(False, y00, aggp0, degp, W1_0, b1_0)
    aggp1 = _sc_spmm()(x1, pk_r)
    y01 = _tc_y0(x1, W0_1, b0_1)
    x2 = _tc_comb(True, y01, aggp1, degp, W1_1, b1_1)
    aggp2 = _sc_spmm()(x2, pk_r)
    y02 = _tc_y0(x2, W0_2, b0_2)
    return _tc_final(y02, aggp2, degp, W1_2, b1_2, features)
